# baseline (speedup 1.0000x reference)
import functools

import jax
import jax.numpy as jnp
from jax import lax
from jax.experimental import pallas as pl
from jax.experimental.pallas import tpu as pltpu
from jax.experimental.pallas import tpu_sc as plsc

B, S, D, V, NSTATES = 32, 8, 1024, 100000, 4
K = 50
CAND = 64
VB = 2048
VPAD = 100352
NC, NS, L = 2, 16, 16
NW = NC * NS
RPW = (B * S) // NW
CH = VPAD // NW
NVR = CH // L
NEG = -3.0e38
BIGI = 2**30

_mesh = plsc.VectorSubcoreMesh(
    core_axis_name="c", subcore_axis_name="s", num_cores=NC, num_subcores=NS)
_sc_params = pltpu.CompilerParams(needs_layout_passes=False)


def _wid():
    return lax.axis_index("s") * NC + lax.axis_index("c")


def _gather_body(ids_hbm, emb_hbm, hsum_hbm, row0_hbm, idx_v, rows_v, sum_v, sem):
    w = _wid()
    base = w * RPW
    pltpu.sync_copy(ids_hbm.at[pl.ds(base, RPW)], idx_v)
    pltpu.async_copy(emb_hbm.at[idx_v], rows_v, sem).wait()
    for d in range(D // L):
        sl = pl.ds(d * L, L)
        acc = rows_v[0, sl]
        for r in range(1, RPW):
            acc = acc + rows_v[r, sl]
        sum_v[sl] = acc
    pltpu.sync_copy(sum_v, hsum_hbm.at[w])

    @pl.when(w == 0)
    def _():
        pltpu.sync_copy(rows_v.at[RPW - 1], row0_hbm)


_gather = pl.kernel(
    _gather_body,
    out_type=(
        jax.ShapeDtypeStruct((B, D), jnp.float32),
        jax.ShapeDtypeStruct((D,), jnp.float32),
    ),
    mesh=_mesh,
    scratch_types=(
        pltpu.VMEM((RPW,), jnp.int32),
        pltpu.VMEM((RPW, D), jnp.float32),
        pltpu.VMEM((D,), jnp.float32),
        pltpu.SemaphoreType.DMA,
    ),
    compiler_params=_sc_params,
)


def _tc_body(states_ref, hsum_ref, row0_ref, u_ref, w_ref, ns_ref, last_ref, h0_scr):
    i = pl.program_id(0)

    @pl.when(i == 0)
    def _():
        st = states_ref[...]
        s_sum = jnp.sum(st, axis=0)
        pooled = hsum_ref[...] * (1.0 / S) + s_sum
        z = jnp.dot(pooled, u_ref[...], preferred_element_type=jnp.float32)
        ns_ref[...] = jnp.tanh(z[None, :, :] + st)
        h0_scr[...] = row0_ref[...] + s_sum[0:1, :]

    blk = jnp.dot(h0_scr[...], w_ref[...], preferred_element_type=jnp.float32)
    col = i * VB + lax.broadcasted_iota(jnp.int32, (1, VB), 1)
    last_ref[...] = jnp.where(col < V, blk, NEG)


_tc = pl.pallas_call(
    _tc_body,
    grid=(VPAD // VB,),
    in_specs=[
        pl.BlockSpec((NSTATES, B, D), lambda i: (0, 0, 0)),
        pl.BlockSpec((B, D), lambda i: (0, 0)),
        pl.BlockSpec((1, D), lambda i: (0, 0)),
        pl.BlockSpec((D, D), lambda i: (0, 0)),
        pl.BlockSpec((D, VB), lambda i: (0, i)),
    ],
    out_specs=[
        pl.BlockSpec((NSTATES, B, D), lambda i: (0, 0, 0)),
        pl.BlockSpec((1, VB), lambda i: (0, i)),
    ],
    out_shape=[
        jax.ShapeDtypeStruct((NSTATES, B, D), jnp.float32),
        jax.ShapeDtypeStruct((1, VPAD), jnp.float32),
    ],
    scratch_shapes=[pltpu.VMEM((1, D), jnp.float32)],
    compiler_params=pltpu.CompilerParams(dimension_semantics=("arbitrary",)),
)


def _topk_rounds(x_v, vals_v, idx_v, nvr, idx_of):
    lane = jnp.arange(L, dtype=jnp.int32)
    for g in range(CAND // L):
        vals_v[pl.ds(g * L, L)] = jnp.full((L,), NEG, jnp.float32)
        idx_v[pl.ds(g * L, L)] = jnp.zeros((L,), jnp.int32)

    def round_body(r, carry):
        def scan_body(j, mc):
            m, jm = mc
            v = x_v[pl.ds(pl.multiple_of(j * L, L), L)]
            upd = v > m
            return jnp.where(upd, v, m), jnp.where(upd, j, jm)

        m0 = jnp.full((L,), NEG, jnp.float32)
        jm0 = jnp.zeros((L,), jnp.int32)
        m, jm = lax.fori_loop(0, nvr, scan_body, (m0, jm0))
        M = jnp.max(m)
        fl = jnp.where(m == M, jm * L + lane, BIGI)
        F = jnp.min(fl)
        g16 = pl.multiple_of((r // L) * L, L)
        sel = lane == (r % L)
        vals_v[pl.ds(g16, L)] = jnp.where(sel, M, vals_v[pl.ds(g16, L)])
        idx_v[pl.ds(g16, L)] = jnp.where(sel, idx_of(F), idx_v[pl.ds(g16, L)])
        rowb = pl.multiple_of((F // L) * L, L)
        x_v[pl.ds(rowb, L)] = jnp.where(
            lane == (F % L), NEG, x_v[pl.ds(rowb, L)])
        return carry

    lax.fori_loop(0, K, round_body, 0)


def _topk_body(last_hbm, cv_hbm, ci_hbm, x_v, vals_v, idx_v):
    w = _wid()
    base = w * CH
    pltpu.sync_copy(last_hbm.at[pl.ds(base, CH)], x_v)
    _topk_rounds(x_v, vals_v, idx_v, NVR, lambda F: base + F)
    pltpu.sync_copy(vals_v, cv_hbm.at[w])
    pltpu.sync_copy(idx_v, ci_hbm.at[w])


_topk = pl.kernel(
    _topk_body,
    out_type=(
        jax.ShapeDtypeStruct((NW, CAND), jnp.float32),
        jax.ShapeDtypeStruct((NW, CAND), jnp.int32),
    ),
    mesh=_mesh,
    scratch_types=(
        pltpu.VMEM((CH,), jnp.float32),
        pltpu.VMEM((CAND,), jnp.float32),
        pltpu.VMEM((CAND,), jnp.int32),
    ),
    compiler_params=_sc_params,
)


def _merge_body(cv_hbm, ci_hbm, e_hbm, out_hbm, x_v, xi_v, vals_v, idx_v, e_v, out_v):
    w = _wid()

    @pl.when(w == 0)
    def _():
        lane = jnp.arange(L, dtype=jnp.int32)
        pltpu.sync_copy(cv_hbm, x_v)
        pltpu.sync_copy(ci_hbm, xi_v)
        pltpu.sync_copy(e_hbm, e_v)
        nvr2 = (NW * CAND) // L

        def idx_of(F):
            rowb = pl.multiple_of((F // L) * L, L)
            iv = xi_v[pl.ds(rowb, L)]
            return jnp.max(jnp.where(lane == (F % L), iv, jnp.int32(-1)))

        _topk_rounds(x_v, vals_v, idx_v, nvr2, idx_of)
        m = jnp.max(vals_v[pl.ds(0, L)])
        ex = [jnp.exp(vals_v[pl.ds(g * L, L)] - m) for g in range(CAND // L)]
        zs = jnp.sum(ex[0] + ex[1] + ex[2] + ex[3])
        sc = [(ex[g] / zs + 1e-9) * e_v[pl.ds(g * L, L)] for g in range(CAND // L)]
        mm = jnp.maximum(jnp.maximum(sc[0], sc[1]), jnp.maximum(sc[2], sc[3]))
        ms = jnp.max(mm)
        fls = [jnp.where(sc[g] == ms, g * L + lane, BIGI) for g in range(CAND // L)]
        pid = jnp.min(jnp.minimum(jnp.minimum(fls[0], fls[1]),
                                  jnp.minimum(fls[2], fls[3])))
        prow = pl.multiple_of((pid // L) * L, L)
        pv = idx_v[pl.ds(prow, L)]
        predicted = jnp.max(jnp.where(lane == (pid % L), pv, jnp.int32(-1)))
        out_v[...] = jnp.full((L,), 0, jnp.int32) + predicted
        pltpu.sync_copy(out_v, out_hbm)


_merge = pl.kernel(
    _merge_body,
    out_type=jax.ShapeDtypeStruct((L,), jnp.int32),
    mesh=_mesh,
    scratch_types=(
        pltpu.VMEM((NW * CAND,), jnp.float32),
        pltpu.VMEM((NW * CAND,), jnp.int32),
        pltpu.VMEM((CAND,), jnp.float32),
        pltpu.VMEM((CAND,), jnp.int32),
        pltpu.VMEM((CAND,), jnp.float32),
        pltpu.VMEM((L,), jnp.int32),
    ),
    compiler_params=_sc_params,
)


def kernel(input_ids, states, emb, W, U, random):
    ids_flat = input_ids.reshape(B * S).astype(jnp.int32)
    hsum, row0 = _gather(ids_flat, emb)
    new_states, last = _tc(states, hsum, row0.reshape(1, D), U, W)
    cand_vals, cand_idx = _topk(last.reshape(VPAD))
    g = jax.random.gumbel(jax.random.key(1), (K,), jnp.float32)
    e_gumbel = jnp.concatenate([jnp.exp(g), jnp.zeros((CAND - K,), jnp.float32)])
    out = _merge(cand_vals.reshape(NW * CAND), cand_idx.reshape(NW * CAND), e_gumbel)
    return (out[0], new_states)
